# baseline (speedup 1.0000x reference)
import numpy as np
import jax
import jax.numpy as jnp
from jax.experimental import pallas as pl
from jax.experimental.pallas import tpu as pltpu

_TEMPERATURE = 0.5
_B = 128
_V = 100000
_ROWS_PER_BLOCK = 8

_ROTS = ((13, 15, 26, 6), (17, 29, 16, 24))


def _np_threefry2x32(k0, k1, x0, x1):
    def rotl(x, d):
        return ((x << np.uint32(d)) | (x >> np.uint32(32 - d))).astype(np.uint32)

    ks = [np.uint32(k0), np.uint32(k1),
          np.uint32(k0 ^ k1 ^ np.uint32(0x1BD11BDA))]
    x0 = (x0 + ks[0]).astype(np.uint32)
    x1 = (x1 + ks[1]).astype(np.uint32)
    for g in range(5):
        for r in _ROTS[g % 2]:
            x0 = (x0 + x1).astype(np.uint32)
            x1 = (x0 ^ rotl(x1, r)).astype(np.uint32)
        x0 = (x0 + ks[(g + 1) % 3]).astype(np.uint32)
        x1 = (x1 + ks[(g + 2) % 3] + np.uint32(g + 1)).astype(np.uint32)
    return x0, x1


_FK0, _FK1 = (int(a[0]) for a in _np_threefry2x32(
    np.uint32(0), np.uint32(0), np.uint32([0]), np.uint32([12345])))
_FKS2 = _FK0 ^ _FK1 ^ 0x1BD11BDA


_CHUNK = 512


def _noise_weight(lin):
    ks = (jnp.uint32(_FK0), jnp.uint32(_FK1), jnp.uint32(_FKS2))
    a = jnp.full(lin.shape, ks[0], dtype=jnp.uint32)
    b = lin + ks[1]
    for g in range(5):
        for r in _ROTS[g % 2]:
            a = a + b
            b = a ^ ((b << r) | (b >> (32 - r)))
        a = a + ks[(g + 1) % 3]
        b = b + ks[(g + 2) % 3] + jnp.uint32(g + 1)
    bits = a ^ b

    fb = (bits >> 9) | jnp.uint32(0x3F800000)
    f = jax.lax.bitcast_convert_type(fb, jnp.float32) - jnp.float32(1.0)
    tiny = jnp.float32(np.finfo(np.float32).tiny)
    u = jnp.maximum(tiny, f * (jnp.float32(1.0) - tiny) + tiny)
    w = -jnp.log(u)
    iw = jnp.float32(1.0) / w
    return iw * iw


def _gumbel_softmax_kernel(x_ref, o_ref):
    rb, v = x_ref.shape
    blk = pl.program_id(0)

    c = (jnp.float32(2.0) * jnp.max(x_ref[...], axis=1, keepdims=True)
         + jnp.float32(34.0))

    rowbase = ((blk * rb + jax.lax.broadcasted_iota(jnp.int32, (rb, _CHUNK), 0))
               * v).astype(jnp.uint32)
    col = jax.lax.broadcasted_iota(jnp.int32, (rb, _CHUNK), 1).astype(jnp.uint32)

    nfull = v // _CHUNK

    def chunk_sum(j, s_acc):
        sl = pl.ds(j * _CHUNK, _CHUNK)
        lin = rowbase + (col + jnp.uint32(j * _CHUNK))
        e = (jnp.exp(jnp.float32(2.0) * x_ref[:, sl] - c) * _noise_weight(lin))
        o_ref[:, sl] = e
        return s_acc + jnp.sum(e, axis=1, keepdims=True)

    s = jax.lax.fori_loop(0, nfull, chunk_sum, jnp.zeros((rb, 1), jnp.float32),
                          unroll=16)

    tail = v - nfull * _CHUNK
    if tail:
        lin_t = (rowbase[:, :tail]
                 + (col[:, :tail] + jnp.uint32(nfull * _CHUNK)))
        e_t = (jnp.exp(jnp.float32(2.0) * x_ref[:, nfull * _CHUNK:] - c)
               * _noise_weight(lin_t))
        o_ref[:, nfull * _CHUNK:] = e_t
        s = s + jnp.sum(e_t, axis=1, keepdims=True)

    o_ref[...] = o_ref[...] * (jnp.float32(1.0) / s)


def kernel(inputs):
    return pl.pallas_call(
        _gumbel_softmax_kernel,
        grid=(_B // _ROWS_PER_BLOCK,),
        in_specs=[pl.BlockSpec((_ROWS_PER_BLOCK, _V), lambda i: (i, 0))],
        out_specs=pl.BlockSpec((_ROWS_PER_BLOCK, _V), lambda i: (i, 0)),
        out_shape=jax.ShapeDtypeStruct((_B, _V), jnp.float32),
        compiler_params=pltpu.CompilerParams(
            dimension_semantics=("parallel",)),
    )(inputs)


# speedup vs baseline: 1.0674x; 1.0674x over previous
import numpy as np
import jax
import jax.numpy as jnp
from jax.experimental import pallas as pl
from jax.experimental.pallas import tpu as pltpu

_TEMPERATURE = 0.5
_B = 128
_V = 100000
_ROWS_PER_BLOCK = 16

_ROTS = ((13, 15, 26, 6), (17, 29, 16, 24))


def _np_threefry2x32(k0, k1, x0, x1):
    def rotl(x, d):
        return ((x << np.uint32(d)) | (x >> np.uint32(32 - d))).astype(np.uint32)

    ks = [np.uint32(k0), np.uint32(k1),
          np.uint32(k0 ^ k1 ^ np.uint32(0x1BD11BDA))]
    x0 = (x0 + ks[0]).astype(np.uint32)
    x1 = (x1 + ks[1]).astype(np.uint32)
    for g in range(5):
        for r in _ROTS[g % 2]:
            x0 = (x0 + x1).astype(np.uint32)
            x1 = (x0 ^ rotl(x1, r)).astype(np.uint32)
        x0 = (x0 + ks[(g + 1) % 3]).astype(np.uint32)
        x1 = (x1 + ks[(g + 2) % 3] + np.uint32(g + 1)).astype(np.uint32)
    return x0, x1


_FK0, _FK1 = (int(a[0]) for a in _np_threefry2x32(
    np.uint32(0), np.uint32(0), np.uint32([0]), np.uint32([12345])))
_FKS2 = _FK0 ^ _FK1 ^ 0x1BD11BDA


_CHUNK = 1024


def _noise_weight(lin):
    ks = (jnp.uint32(_FK0), jnp.uint32(_FK1), jnp.uint32(_FKS2))
    a = jnp.full(lin.shape, ks[0], dtype=jnp.uint32)
    b = lin + ks[1]
    for g in range(5):
        for r in _ROTS[g % 2]:
            a = a + b
            b = a ^ ((b << r) | (b >> (32 - r)))
        a = a + ks[(g + 1) % 3]
        b = b + ks[(g + 2) % 3] + jnp.uint32(g + 1)
    bits = a ^ b

    fb = (bits >> 9) | jnp.uint32(0x3F800000)
    f = jax.lax.bitcast_convert_type(fb, jnp.float32) - jnp.float32(1.0)
    tiny = jnp.float32(np.finfo(np.float32).tiny)
    u = jnp.maximum(tiny, f * (jnp.float32(1.0) - tiny) + tiny)
    w = -jnp.log(u)
    iw = jnp.float32(1.0) / w
    return iw * iw


def _gumbel_softmax_kernel(x_ref, o_ref):
    rb, v = x_ref.shape
    blk = pl.program_id(0)

    c = (jnp.float32(2.0) * jnp.max(x_ref[...], axis=1, keepdims=True)
         + jnp.float32(34.0))

    rowbase = ((blk * rb + jax.lax.broadcasted_iota(jnp.int32, (rb, _CHUNK), 0))
               * v).astype(jnp.uint32)
    col = jax.lax.broadcasted_iota(jnp.int32, (rb, _CHUNK), 1).astype(jnp.uint32)

    nfull = v // _CHUNK

    def chunk_sum(j, s_acc):
        sl = pl.ds(j * _CHUNK, _CHUNK)
        lin = rowbase + (col + jnp.uint32(j * _CHUNK))
        e = (jnp.exp(jnp.float32(2.0) * x_ref[:, sl] - c) * _noise_weight(lin))
        o_ref[:, sl] = e
        return s_acc + jnp.sum(e, axis=1, keepdims=True)

    s = jax.lax.fori_loop(0, nfull, chunk_sum, jnp.zeros((rb, 1), jnp.float32),
                          unroll=16)

    tail = v - nfull * _CHUNK
    if tail:
        lin_t = (rowbase[:, :tail]
                 + (col[:, :tail] + jnp.uint32(nfull * _CHUNK)))
        e_t = (jnp.exp(jnp.float32(2.0) * x_ref[:, nfull * _CHUNK:] - c)
               * _noise_weight(lin_t))
        o_ref[:, nfull * _CHUNK:] = e_t
        s = s + jnp.sum(e_t, axis=1, keepdims=True)

    o_ref[...] = o_ref[...] * (jnp.float32(1.0) / s)


def kernel(inputs):
    return pl.pallas_call(
        _gumbel_softmax_kernel,
        grid=(_B // _ROWS_PER_BLOCK,),
        in_specs=[pl.BlockSpec((_ROWS_PER_BLOCK, _V), lambda i: (i, 0))],
        out_specs=pl.BlockSpec((_ROWS_PER_BLOCK, _V), lambda i: (i, 0)),
        out_shape=jax.ShapeDtypeStruct((_B, _V), jnp.float32),
        compiler_params=pltpu.CompilerParams(
            dimension_semantics=("parallel",)),
    )(inputs)


# speedup vs baseline: 1.0713x; 1.0036x over previous
import numpy as np
import jax
import jax.numpy as jnp
from jax.experimental import pallas as pl
from jax.experimental.pallas import tpu as pltpu

_TEMPERATURE = 0.5
_B = 128
_V = 100000
_ROWS_PER_BLOCK = 32

_ROTS = ((13, 15, 26, 6), (17, 29, 16, 24))


def _np_threefry2x32(k0, k1, x0, x1):
    def rotl(x, d):
        return ((x << np.uint32(d)) | (x >> np.uint32(32 - d))).astype(np.uint32)

    ks = [np.uint32(k0), np.uint32(k1),
          np.uint32(k0 ^ k1 ^ np.uint32(0x1BD11BDA))]
    x0 = (x0 + ks[0]).astype(np.uint32)
    x1 = (x1 + ks[1]).astype(np.uint32)
    for g in range(5):
        for r in _ROTS[g % 2]:
            x0 = (x0 + x1).astype(np.uint32)
            x1 = (x0 ^ rotl(x1, r)).astype(np.uint32)
        x0 = (x0 + ks[(g + 1) % 3]).astype(np.uint32)
        x1 = (x1 + ks[(g + 2) % 3] + np.uint32(g + 1)).astype(np.uint32)
    return x0, x1


_FK0, _FK1 = (int(a[0]) for a in _np_threefry2x32(
    np.uint32(0), np.uint32(0), np.uint32([0]), np.uint32([12345])))
_FKS2 = _FK0 ^ _FK1 ^ 0x1BD11BDA


_CHUNK = 1024


def _noise_weight(lin):
    ks = (jnp.uint32(_FK0), jnp.uint32(_FK1), jnp.uint32(_FKS2))
    a = jnp.full(lin.shape, ks[0], dtype=jnp.uint32)
    b = lin + ks[1]
    for g in range(5):
        for r in _ROTS[g % 2]:
            a = a + b
            b = a ^ ((b << r) | (b >> (32 - r)))
        a = a + ks[(g + 1) % 3]
        b = b + ks[(g + 2) % 3] + jnp.uint32(g + 1)
    bits = a ^ b

    fb = (bits >> 9) | jnp.uint32(0x3F800000)
    f = jax.lax.bitcast_convert_type(fb, jnp.float32) - jnp.float32(1.0)
    tiny = jnp.float32(np.finfo(np.float32).tiny)
    u = jnp.maximum(tiny, f * (jnp.float32(1.0) - tiny) + tiny)
    w = -jnp.log(u)
    iw = jnp.float32(1.0) / w
    return iw * iw


def _gumbel_softmax_kernel(x_ref, o_ref):
    rb, v = x_ref.shape
    blk = pl.program_id(0)

    c = (jnp.float32(2.0) * jnp.max(x_ref[...], axis=1, keepdims=True)
         + jnp.float32(34.0))

    rowbase = ((blk * rb + jax.lax.broadcasted_iota(jnp.int32, (rb, _CHUNK), 0))
               * v).astype(jnp.uint32)
    col = jax.lax.broadcasted_iota(jnp.int32, (rb, _CHUNK), 1).astype(jnp.uint32)

    nfull = v // _CHUNK

    def chunk_sum(j, s_acc):
        sl = pl.ds(j * _CHUNK, _CHUNK)
        lin = rowbase + (col + jnp.uint32(j * _CHUNK))
        e = (jnp.exp(jnp.float32(2.0) * x_ref[:, sl] - c) * _noise_weight(lin))
        o_ref[:, sl] = e
        return s_acc + jnp.sum(e, axis=1, keepdims=True)

    s = jax.lax.fori_loop(0, nfull, chunk_sum, jnp.zeros((rb, 1), jnp.float32),
                          unroll=16)

    tail = v - nfull * _CHUNK
    if tail:
        lin_t = (rowbase[:, :tail]
                 + (col[:, :tail] + jnp.uint32(nfull * _CHUNK)))
        e_t = (jnp.exp(jnp.float32(2.0) * x_ref[:, nfull * _CHUNK:] - c)
               * _noise_weight(lin_t))
        o_ref[:, nfull * _CHUNK:] = e_t
        s = s + jnp.sum(e_t, axis=1, keepdims=True)

    o_ref[...] = o_ref[...] * (jnp.float32(1.0) / s)


def kernel(inputs):
    return pl.pallas_call(
        _gumbel_softmax_kernel,
        grid=(_B // _ROWS_PER_BLOCK,),
        in_specs=[pl.BlockSpec((_ROWS_PER_BLOCK, _V), lambda i: (i, 0))],
        out_specs=pl.BlockSpec((_ROWS_PER_BLOCK, _V), lambda i: (i, 0)),
        out_shape=jax.ShapeDtypeStruct((_B, _V), jnp.float32),
        compiler_params=pltpu.CompilerParams(
            dimension_semantics=("parallel",)),
    )(inputs)


# speedup vs baseline: 1.0805x; 1.0086x over previous
import numpy as np
import jax
import jax.numpy as jnp
from jax.experimental import pallas as pl
from jax.experimental.pallas import tpu as pltpu

_TEMPERATURE = 0.5
_B = 128
_V = 100000
_ROWS_PER_BLOCK = 32

_ROTS = ((13, 15, 26, 6), (17, 29, 16, 24))


def _np_threefry2x32(k0, k1, x0, x1):
    def rotl(x, d):
        return ((x << np.uint32(d)) | (x >> np.uint32(32 - d))).astype(np.uint32)

    ks = [np.uint32(k0), np.uint32(k1),
          np.uint32(k0 ^ k1 ^ np.uint32(0x1BD11BDA))]
    x0 = (x0 + ks[0]).astype(np.uint32)
    x1 = (x1 + ks[1]).astype(np.uint32)
    for g in range(5):
        for r in _ROTS[g % 2]:
            x0 = (x0 + x1).astype(np.uint32)
            x1 = (x0 ^ rotl(x1, r)).astype(np.uint32)
        x0 = (x0 + ks[(g + 1) % 3]).astype(np.uint32)
        x1 = (x1 + ks[(g + 2) % 3] + np.uint32(g + 1)).astype(np.uint32)
    return x0, x1


_FK0, _FK1 = (int(a[0]) for a in _np_threefry2x32(
    np.uint32(0), np.uint32(0), np.uint32([0]), np.uint32([12345])))
_FKS2 = _FK0 ^ _FK1 ^ 0x1BD11BDA


_CHUNK = 1024


def _noise_weight(lin):
    ks = (jnp.uint32(_FK0), jnp.uint32(_FK1), jnp.uint32(_FKS2))
    a = jnp.full(lin.shape, ks[0], dtype=jnp.uint32)
    b = lin + ks[1]
    for g in range(5):
        for r in _ROTS[g % 2]:
            a = a + b
            b = a ^ ((b << r) | (b >> (32 - r)))
        a = a + ks[(g + 1) % 3]
        b = b + ks[(g + 2) % 3] + jnp.uint32(g + 1)
    bits = a ^ b

    fb = (bits >> 9) | jnp.uint32(0x3F800000)
    f = jax.lax.bitcast_convert_type(fb, jnp.float32) - jnp.float32(1.0)
    tiny = jnp.float32(np.finfo(np.float32).tiny)
    u = jnp.maximum(tiny, f * (jnp.float32(1.0) - tiny) + tiny)
    w = -jnp.log(u)
    iw = jnp.float32(1.0) / w
    return iw * iw


def _gumbel_softmax_kernel(x_ref, o_ref):
    rb, v = x_ref.shape
    blk = pl.program_id(0)

    c = (jnp.float32(2.0) * jnp.max(x_ref[...], axis=1, keepdims=True)
         + jnp.float32(34.0))

    rowbase = ((blk * rb + jax.lax.broadcasted_iota(jnp.int32, (rb, _CHUNK), 0))
               * v).astype(jnp.uint32)
    col = jax.lax.broadcasted_iota(jnp.int32, (rb, _CHUNK), 1).astype(jnp.uint32)

    nfull = v // _CHUNK

    def chunk_sum(j, carry):
        lin, s_acc = carry
        sl = pl.ds(j * _CHUNK, _CHUNK)
        e = (jnp.exp(jnp.float32(2.0) * x_ref[:, sl] - c) * _noise_weight(lin))
        o_ref[:, sl] = e
        return lin + jnp.uint32(_CHUNK), s_acc + e

    _, s_vec = jax.lax.fori_loop(
        0, nfull, chunk_sum,
        (rowbase + col, jnp.zeros((rb, _CHUNK), jnp.float32)),
        unroll=16)
    s = jnp.sum(s_vec, axis=1, keepdims=True)

    tail = v - nfull * _CHUNK
    if tail:
        lin_t = (rowbase[:, :tail]
                 + (col[:, :tail] + jnp.uint32(nfull * _CHUNK)))
        e_t = (jnp.exp(jnp.float32(2.0) * x_ref[:, nfull * _CHUNK:] - c)
               * _noise_weight(lin_t))
        o_ref[:, nfull * _CHUNK:] = e_t
        s = s + jnp.sum(e_t, axis=1, keepdims=True)

    o_ref[...] = o_ref[...] * (jnp.float32(1.0) / s)


def kernel(inputs):
    return pl.pallas_call(
        _gumbel_softmax_kernel,
        grid=(_B // _ROWS_PER_BLOCK,),
        in_specs=[pl.BlockSpec((_ROWS_PER_BLOCK, _V), lambda i: (i, 0))],
        out_specs=pl.BlockSpec((_ROWS_PER_BLOCK, _V), lambda i: (i, 0)),
        out_shape=jax.ShapeDtypeStruct((_B, _V), jnp.float32),
        compiler_params=pltpu.CompilerParams(
            dimension_semantics=("parallel",)),
    )(inputs)
